# Initial kernel scaffold; baseline (speedup 1.0000x reference)
import jax
import jax.numpy as jnp
from jax import lax
from jax.experimental import pallas as pl
from jax.experimental.pallas import tpu as pltpu
from jax.experimental.pallas import tpu_sc as plsc

NSEG = 10000
TBL = 10240
NC, NS = 2, 16
ROWS = 320000
D = 128
PER_SUB = ROWS // NS
TILE = 80
N_TILES = PER_SUB // TILE
SLICE = TBL // NS


def _square_body(x_ref, o_ref):
    x = x_ref[...]
    o_ref[...] = x * x


def _tc_square(data):
    blk = 2500
    return pl.pallas_call(
        _square_body,
        grid=(ROWS // blk,),
        in_specs=[pl.BlockSpec((blk, D), lambda i: (i, 0))],
        out_specs=pl.BlockSpec((blk, D), lambda i: (i, 0)),
        out_shape=jax.ShapeDtypeStruct((ROWS, D), jnp.float32),
    )(data)


def _sc_body(data_hbm, sq_hbm, ids_hbm, zrow_hbm, zcnt_hbm,
             cnt_hbm, sums_hbm, sqs_hbm,
             tile_a, tile_b, ids_a, ids_b, ones_v,
             acc_sh, cnt_sh, sem_a, sem_b):
    c = lax.axis_index("c")
    s = lax.axis_index("s")
    base = s * PER_SUB
    off = s * SLICE

    def issue(src_hbm, t, tile, idsb, sem):
        start = base + t * TILE
        pltpu.async_copy(src_hbm.at[pl.ds(start, TILE)], tile, sem)
        pltpu.async_copy(ids_hbm.at[pl.ds(start, TILE)], idsb.at[0], sem)

    def wait(src_hbm, t, tile, idsb, sem):
        start = base + t * TILE
        pltpu.make_async_copy(src_hbm.at[pl.ds(start, TILE)], tile, sem).wait()
        pltpu.make_async_copy(
            ids_hbm.at[pl.ds(start, TILE)], idsb.at[0], sem).wait()

    def scatter(tile, idsb, do_cnt):
        pltpu.sync_copy(tile, acc_sh.at[idsb.at[0]], add=True)
        if do_cnt:
            pltpu.sync_copy(ones_v, cnt_sh.at[idsb.at[0]], add=True)

    def stream(src_hbm, do_cnt):
        issue(src_hbm, 0, tile_a, ids_a, sem_a)

        pltpu.sync_copy(zrow_hbm, acc_sh.at[pl.ds(off, SLICE)])
        pltpu.sync_copy(zcnt_hbm, cnt_sh.at[pl.ds(off, SLICE)])

        if do_cnt:
            @pl.loop(0, TILE)
            def _(r):
                ones_v[r, :] = jnp.ones((16,), jnp.float32)

        plsc.subcore_barrier()

        @pl.loop(0, N_TILES, step=2)
        def _(t):
            issue(src_hbm, t + 1, tile_b, ids_b, sem_b)
            wait(src_hbm, t, tile_a, ids_a, sem_a)
            scatter(tile_a, ids_a, do_cnt)

            @pl.when(t + 2 < N_TILES)
            def _():
                issue(src_hbm, t + 2, tile_a, ids_a, sem_a)

            wait(src_hbm, t + 1, tile_b, ids_b, sem_b)
            scatter(tile_b, ids_b, do_cnt)

        plsc.subcore_barrier()

    @pl.when(c == 0)
    def _():
        stream(data_hbm, True)
        pltpu.sync_copy(acc_sh.at[pl.ds(off, SLICE)],
                        sums_hbm.at[pl.ds(off, SLICE)])
        pltpu.sync_copy(cnt_sh.at[pl.ds(off, SLICE)],
                        cnt_hbm.at[pl.ds(off, SLICE)])

    @pl.when(c == 1)
    def _():
        stream(sq_hbm, False)
        pltpu.sync_copy(acc_sh.at[pl.ds(off, SLICE)],
                        sqs_hbm.at[pl.ds(off, SLICE)])


def _sc_accumulate(data, sq, ids32, zrow, zcnt):
    mesh = plsc.VectorSubcoreMesh(core_axis_name="c", subcore_axis_name="s")
    f32 = jnp.float32
    return pl.kernel(
        _sc_body,
        out_type=[
            jax.ShapeDtypeStruct((TBL, 16), f32),
            jax.ShapeDtypeStruct((TBL, D), f32),
            jax.ShapeDtypeStruct((TBL, D), f32),
        ],
        mesh=mesh,
        scratch_types=[
            pltpu.VMEM((TILE, D), f32),
            pltpu.VMEM((TILE, D), f32),
            pltpu.VMEM((1, TILE), jnp.int32),
            pltpu.VMEM((1, TILE), jnp.int32),
            pltpu.VMEM((TILE, 16), f32),
            pltpu.VMEM_SHARED((TBL, D), f32),
            pltpu.VMEM_SHARED((TBL, 16), f32),
            pltpu.SemaphoreType.DMA,
            pltpu.SemaphoreType.DMA,
        ],
    )(data, sq, ids32, zrow, zcnt)


def _finalize_body(cnt_ref, sums_ref, sqs_ref, o_ref):
    c = cnt_ref[:, 0:1]
    cs = jnp.maximum(c, 1.0)
    mean = sums_ref[...] / cs
    ex2 = sqs_ref[...] / cs
    var_b = ex2 - mean * mean
    corr = c / jnp.maximum(c - 1.0, 1.0)
    var_u = var_b * corr
    sem = jnp.sqrt(jnp.maximum(var_u, 1e-12) / cs)
    o_ref[:, 0:D] = mean
    o_ref[:, D:2 * D] = sem


def _tc_finalize(cnt, sums, sqs):
    blk = 1280
    return pl.pallas_call(
        _finalize_body,
        grid=(TBL // blk,),
        in_specs=[
            pl.BlockSpec((blk, 16), lambda i: (i, 0)),
            pl.BlockSpec((blk, D), lambda i: (i, 0)),
            pl.BlockSpec((blk, D), lambda i: (i, 0)),
        ],
        out_specs=pl.BlockSpec((blk, 2 * D), lambda i: (i, 0)),
        out_shape=jax.ShapeDtypeStruct((TBL, 2 * D), jnp.float32),
    )(cnt, sums, sqs)


@jax.jit
def _impl(data, segment_ids):
    ids32 = segment_ids.astype(jnp.int32)
    sq = _tc_square(data)
    zrow = jnp.zeros((SLICE, D), jnp.float32)
    zcnt = jnp.zeros((SLICE, 16), jnp.float32)
    cnt, sums, sqs = _sc_accumulate(data, sq, ids32, zrow, zcnt)
    table = _tc_finalize(cnt, sums, sqs)
    return table[:NSEG]


def kernel(data, segment_ids):
    return _impl(data, segment_ids)


# speedup vs baseline: 5.7553x; 5.7553x over previous
import dataclasses

import jax
import jax.numpy as jnp
from jax import lax
from jax.experimental import pallas as pl
from jax.experimental.pallas import tpu as pltpu
from jax.experimental.pallas import tpu_sc as plsc

NSEG = 10000
TBL = 10240
NC, NS = 2, 16
ROWS = 320000
D = 128
EXT = D + 16
PER_SUB = ROWS // NS
TILE = 80
N_TILES = PER_SUB // TILE
SLICE = TBL // NS


def _square_body(x_ref, o_ref):
    x = x_ref[...]
    o_ref[...] = x * x


def _tc_square(data):
    blk = 2560
    return pl.pallas_call(
        _square_body,
        grid=(ROWS // blk,),
        in_specs=[pl.BlockSpec((blk, D), lambda i: (i, 0))],
        out_specs=pl.BlockSpec((blk, D), lambda i: (i, 0)),
        out_shape=jax.ShapeDtypeStruct((ROWS, D), jnp.float32),
    )(data)


def _sc_body(data_hbm, sq_hbm, ids_hbm,
             sums_hbm, sqs_hbm, cnt_hbm,
             tile_a, tile_b, ids_a, ids_b, cnt_ts,
             acc_sh, sem_a, sem_b):
    c = lax.axis_index("c")
    s = lax.axis_index("s")
    base = s * PER_SUB
    off = s * SLICE
    zvec = jnp.zeros((16,), jnp.float32)

    def issue(src_hbm, t, tile, idsb, sem):
        start = base + t * TILE
        pltpu.async_copy(src_hbm.at[pl.ds(start, TILE)], tile, sem)
        pltpu.async_copy(ids_hbm.at[pl.ds(start, TILE)], idsb.at[0], sem)

    def wait(src_hbm, t, tile, idsb, sem):
        start = base + t * TILE
        pltpu.make_async_copy(src_hbm.at[pl.ds(start, TILE)], tile, sem).wait()
        pltpu.make_async_copy(
            ids_hbm.at[pl.ds(start, TILE)], idsb.at[0], sem).wait()

    def scatter(tile, idsb):
        pltpu.sync_copy(tile, acc_sh.at[idsb.at[0]], add=True)

    iot = lax.iota(jnp.int32, 16)
    rot_prev = (iot + 15) & 15
    rot_next = (iot + 1) & 15

    def count(idsb):
        @pl.loop(0, TILE // 16)
        def _(k):
            v = idsb[0, pl.ds(k * 16, 16)]
            prev = jnp.where(
                iot == 0, -1, v.at[rot_prev].get(mode="promise_in_bounds"))
            nxt = v.at[rot_next].get(mode="promise_in_bounds")
            is_first = prev != v
            is_last = jnp.logical_or(iot == 15, v != nxt)
            run_start = plsc.cummax(jnp.where(is_first, iot, 0))
            run_len = (iot - run_start + 1).astype(jnp.float32)
            plsc.addupdate_scatter(cnt_ts, [v], run_len, mask=is_last)

    def stream(src_hbm, do_cnt):
        if N_TILES:
            issue(src_hbm, 0, tile_a, ids_a, sem_a)

        @pl.loop(0, N_TILES, step=2)
        def _(t):
            issue(src_hbm, t + 1, tile_b, ids_b, sem_b)
            wait(src_hbm, t, tile_a, ids_a, sem_a)
            scatter(tile_a, ids_a)
            if do_cnt:
                count(ids_a)

            @pl.when(t + 2 < N_TILES)
            def _():
                issue(src_hbm, t + 2, tile_a, ids_a, sem_a)

            wait(src_hbm, t + 1, tile_b, ids_b, sem_b)
            scatter(tile_b, ids_b)
            if do_cnt:
                count(ids_b)

    @pl.loop(0, TILE)
    def _(r):
        for dcol in range(D // 16):
            tile_a[r, pl.ds(dcol * 16, 16)] = zvec

    @pl.loop(0, TBL // 16)
    def _(k):
        cnt_ts[pl.ds(k * 16, 16)] = zvec

    @pl.loop(0, SLICE // TILE)
    def _(k):
        pltpu.sync_copy(tile_a, acc_sh.at[pl.ds(off + k * TILE, TILE)])

    plsc.subcore_barrier()

    @pl.when(c == 0)
    def _():
        stream(data_hbm, True)

    @pl.when(c == 1)
    def _():
        stream(sq_hbm, False)

    plsc.subcore_barrier()

    def write_out(dst_hbm):
        @pl.loop(0, SLICE // TILE)
        def _(k):
            pltpu.sync_copy(acc_sh.at[pl.ds(off + k * TILE, TILE)], tile_a)
            pltpu.sync_copy(tile_a, dst_hbm.at[pl.ds(off + k * TILE, TILE)])

    @pl.when(c == 0)
    def _():
        write_out(sums_hbm)
        pltpu.sync_copy(cnt_ts, cnt_hbm.at[s])

    @pl.when(c == 1)
    def _():
        write_out(sqs_hbm)


def _sc_accumulate(data, sq, ids32):
    mesh = plsc.VectorSubcoreMesh(core_axis_name="c", subcore_axis_name="s")
    f32 = jnp.float32
    cp = pltpu.CompilerParams()
    if "needs_layout_passes" in pltpu.CompilerParams.__dataclass_fields__:
        cp = dataclasses.replace(cp, needs_layout_passes=False)
    return pl.kernel(
        _sc_body,
        out_type=[
            jax.ShapeDtypeStruct((TBL, D), f32),
            jax.ShapeDtypeStruct((TBL, D), f32),
            jax.ShapeDtypeStruct((NS, TBL), f32),
        ],
        mesh=mesh,
        scratch_types=[
            pltpu.VMEM((TILE, D), f32),
            pltpu.VMEM((TILE, D), f32),
            pltpu.VMEM((1, TILE), jnp.int32),
            pltpu.VMEM((1, TILE), jnp.int32),
            pltpu.VMEM((TBL,), f32),
            pltpu.VMEM_SHARED((TBL, D), f32),
            pltpu.SemaphoreType.DMA,
            pltpu.SemaphoreType.DMA,
        ],
        compiler_params=cp,
    )(data, sq, ids32)


def _finalize_body(sums_ref, sqs_ref, cnt_ref, o_ref):
    c = jnp.sum(cnt_ref[...], axis=0)[:, None]
    cs = jnp.maximum(c, 1.0)
    mean = sums_ref[...] / cs
    ex2 = sqs_ref[...] / cs
    var_b = ex2 - mean * mean
    corr = c / jnp.maximum(c - 1.0, 1.0)
    var_u = var_b * corr
    sem = jnp.sqrt(jnp.maximum(var_u, 1e-12) / cs)
    o_ref[:, 0:D] = mean
    o_ref[:, D:2 * D] = sem


def _tc_finalize(sums, sqs, cnt):
    blk = 1280
    return pl.pallas_call(
        _finalize_body,
        grid=(TBL // blk,),
        in_specs=[
            pl.BlockSpec((blk, D), lambda i: (i, 0)),
            pl.BlockSpec((blk, D), lambda i: (i, 0)),
            pl.BlockSpec((NS, blk), lambda i: (0, i)),
        ],
        out_specs=pl.BlockSpec((blk, 2 * D), lambda i: (i, 0)),
        out_shape=jax.ShapeDtypeStruct((TBL, 2 * D), jnp.float32),
    )(sums, sqs, cnt)


@jax.jit
def _impl(data, segment_ids):
    ids32 = segment_ids.astype(jnp.int32)
    sq = _tc_square(data)
    sums, sqs, cnt = _sc_accumulate(data, sq, ids32)
    table = _tc_finalize(sums, sqs, cnt)
    return table[:NSEG]


def kernel(data, segment_ids):
    return _impl(data, segment_ids)


# speedup vs baseline: 7.0351x; 1.2224x over previous
import dataclasses

import jax
import jax.numpy as jnp
from jax import lax
from jax.experimental import pallas as pl
from jax.experimental.pallas import tpu as pltpu
from jax.experimental.pallas import tpu_sc as plsc

NSEG = 10000
TBL = 10240
NC, NS = 2, 16
ROWS = 320000
D = 128
PER_CORE = ROWS // NC
PER_SUB = PER_CORE // NS
TILE = 80
N_TILES = PER_SUB // TILE
SLICE = TBL // NS


def _square_body(x_ref, o_ref):
    x = x_ref[...]
    o_ref[...] = x * x


def _tc_square(data):
    blk = 2560
    return pl.pallas_call(
        _square_body,
        grid=(ROWS // blk,),
        in_specs=[pl.BlockSpec((blk, D), lambda i: (i, 0))],
        out_specs=pl.BlockSpec((blk, D), lambda i: (i, 0)),
        out_shape=jax.ShapeDtypeStruct((ROWS, D), jnp.float32),
    )(data)


def _make_sc_body(do_cnt):
    def body(src_hbm, ids_hbm, *refs):
        if do_cnt:
            (out0_hbm, out1_hbm, cnt_hbm,
             tile_a, tile_b, ids_a, ids_b, cnt_ts, acc_sh, sem_a, sem_b) = refs
        else:
            (out0_hbm, out1_hbm,
             tile_a, tile_b, ids_a, ids_b, acc_sh, sem_a, sem_b) = refs
        c = lax.axis_index("c")
        s = lax.axis_index("s")
        base = c * PER_CORE + s * PER_SUB
        off = s * SLICE
        zvec = jnp.zeros((16,), jnp.float32)

        def issue(t, tile, idsb, sem):
            start = base + t * TILE
            pltpu.async_copy(src_hbm.at[pl.ds(start, TILE)], tile, sem)
            pltpu.async_copy(ids_hbm.at[pl.ds(start, TILE)], idsb.at[0], sem)

        def wait(t, tile, idsb, sem):
            start = base + t * TILE
            pltpu.make_async_copy(
                src_hbm.at[pl.ds(start, TILE)], tile, sem).wait()
            pltpu.make_async_copy(
                ids_hbm.at[pl.ds(start, TILE)], idsb.at[0], sem).wait()

        def scatter(tile, idsb):
            pltpu.sync_copy(tile, acc_sh.at[idsb.at[0]], add=True)

        iot = lax.iota(jnp.int32, 16)
        rot_prev = (iot + 15) & 15
        rot_next = (iot + 1) & 15

        def count(idsb):
            @pl.loop(0, TILE // 16)
            def _(k):
                v = idsb[0, pl.ds(k * 16, 16)]
                prev = jnp.where(
                    iot == 0, -1,
                    v.at[rot_prev].get(mode="promise_in_bounds"))
                nxt = v.at[rot_next].get(mode="promise_in_bounds")
                is_first = prev != v
                is_last = jnp.logical_or(iot == 15, v != nxt)
                run_start = plsc.cummax(jnp.where(is_first, iot, 0))
                run_len = (iot - run_start + 1).astype(jnp.float32)
                plsc.addupdate_scatter(cnt_ts, [v], run_len, mask=is_last)

        @pl.loop(0, TILE)
        def _(r):
            for dcol in range(D // 16):
                tile_a[r, pl.ds(dcol * 16, 16)] = zvec

        if do_cnt:
            @pl.loop(0, TBL // 16)
            def _(k):
                cnt_ts[pl.ds(k * 16, 16)] = zvec

        @pl.loop(0, SLICE // TILE)
        def _(k):
            pltpu.sync_copy(tile_a, acc_sh.at[pl.ds(off + k * TILE, TILE)])

        plsc.subcore_barrier()

        issue(0, tile_a, ids_a, sem_a)

        @pl.loop(0, N_TILES, step=2)
        def _(t):
            issue(t + 1, tile_b, ids_b, sem_b)
            wait(t, tile_a, ids_a, sem_a)
            scatter(tile_a, ids_a)
            if do_cnt:
                count(ids_a)

            @pl.when(t + 2 < N_TILES)
            def _():
                issue(t + 2, tile_a, ids_a, sem_a)

            wait(t + 1, tile_b, ids_b, sem_b)
            scatter(tile_b, ids_b)
            if do_cnt:
                count(ids_b)

        plsc.subcore_barrier()

        def write_out(dst_hbm):
            @pl.loop(0, SLICE // TILE)
            def _(k):
                pltpu.sync_copy(acc_sh.at[pl.ds(off + k * TILE, TILE)], tile_a)
                pltpu.sync_copy(tile_a, dst_hbm.at[pl.ds(off + k * TILE, TILE)])

        @pl.when(c == 0)
        def _():
            write_out(out0_hbm)

        @pl.when(c == 1)
        def _():
            write_out(out1_hbm)

        if do_cnt:
            pltpu.sync_copy(cnt_ts, cnt_hbm.at[c * NS + s])

    return body


def _sc_scatter(src, ids32, do_cnt):
    mesh = plsc.VectorSubcoreMesh(core_axis_name="c", subcore_axis_name="s")
    f32 = jnp.float32
    cp = pltpu.CompilerParams()
    if "needs_layout_passes" in pltpu.CompilerParams.__dataclass_fields__:
        cp = dataclasses.replace(cp, needs_layout_passes=False)
    out_type = [
        jax.ShapeDtypeStruct((TBL, D), f32),
        jax.ShapeDtypeStruct((TBL, D), f32),
    ]
    scratch = [
        pltpu.VMEM((TILE, D), f32),
        pltpu.VMEM((TILE, D), f32),
        pltpu.VMEM((1, TILE), jnp.int32),
        pltpu.VMEM((1, TILE), jnp.int32),
    ]
    if do_cnt:
        out_type.append(jax.ShapeDtypeStruct((NC * NS, TBL), f32))
        scratch.append(pltpu.VMEM((TBL,), f32))
    scratch += [
        pltpu.VMEM_SHARED((TBL, D), f32),
        pltpu.SemaphoreType.DMA,
        pltpu.SemaphoreType.DMA,
    ]
    return pl.kernel(
        _make_sc_body(do_cnt),
        out_type=out_type,
        mesh=mesh,
        scratch_types=scratch,
        compiler_params=cp,
    )(src, ids32)


def _finalize_body(s0_ref, s1_ref, q0_ref, q1_ref, cnt_ref, o_ref):
    c = jnp.sum(cnt_ref[...], axis=0)[:, None]
    cs = jnp.maximum(c, 1.0)
    mean = (s0_ref[...] + s1_ref[...]) / cs
    ex2 = (q0_ref[...] + q1_ref[...]) / cs
    var_b = ex2 - mean * mean
    corr = c / jnp.maximum(c - 1.0, 1.0)
    var_u = var_b * corr
    sem = jnp.sqrt(jnp.maximum(var_u, 1e-12) / cs)
    o_ref[:, 0:D] = mean
    o_ref[:, D:2 * D] = sem


def _tc_finalize(s0, s1, q0, q1, cnt):
    blk = 1280
    return pl.pallas_call(
        _finalize_body,
        grid=(TBL // blk,),
        in_specs=[
            pl.BlockSpec((blk, D), lambda i: (i, 0)),
            pl.BlockSpec((blk, D), lambda i: (i, 0)),
            pl.BlockSpec((blk, D), lambda i: (i, 0)),
            pl.BlockSpec((blk, D), lambda i: (i, 0)),
            pl.BlockSpec((NC * NS, blk), lambda i: (0, i)),
        ],
        out_specs=pl.BlockSpec((blk, 2 * D), lambda i: (i, 0)),
        out_shape=jax.ShapeDtypeStruct((TBL, 2 * D), jnp.float32),
    )(s0, s1, q0, q1, cnt)


@jax.jit
def _impl(data, segment_ids):
    ids32 = segment_ids.astype(jnp.int32)
    sq = _tc_square(data)
    s0, s1, cnt = _sc_scatter(data, ids32, True)
    q0, q1 = _sc_scatter(sq, ids32, False)
    table = _tc_finalize(s0, s1, q0, q1, cnt)
    return table[:NSEG]


def kernel(data, segment_ids):
    return _impl(data, segment_ids)
